# baseline (speedup 1.0000x reference)
import functools

import jax
import jax.numpy as jnp
from jax import lax
from jax.experimental import pallas as pl
from jax.experimental.pallas import tpu as pltpu
from jax.experimental.pallas import tpu_sc as plsc

DP = 16
K = 4
CHUNK = 1024
VBLK = 512


@functools.lru_cache(maxsize=None)
def _make_detile(vocab: int, d: int):
    info = plsc.get_sparse_core_info()
    nw = info.num_cores * info.num_subcores
    vmain = (vocab // VBLK) * VBLK
    tail = vocab - vmain
    n_blocks = vmain // VBLK
    rows_out = vocab * DP // 128
    rows_blk = VBLK * DP // 128

    mesh = plsc.VectorSubcoreMesh(core_axis_name="c", subcore_axis_name="s")

    @functools.partial(
        pl.kernel,
        mesh=mesh,
        compiler_params=pltpu.CompilerParams(use_tc_tiling_on_sc=True,
                                             needs_layout_passes=False),
        out_type=jax.ShapeDtypeStruct((rows_out, 128), jnp.float32),
        scratch_types=[
            pltpu.VMEM((2, d, VBLK), jnp.float32),
            pltpu.VMEM((2, rows_blk, 128), jnp.float32),
            pltpu.VMEM((tail * DP // 128, 128), jnp.float32),
            [pltpu.SemaphoreType.DMA] * 2,
            [pltpu.SemaphoreType.DMA] * 2,
        ],
    )
    def detile_kernel(tt_hbm, tail_hbm, out_hbm, in_v, out_v, tail_v,
                      isems, osems):
        wid = lax.axis_index("s") * info.num_cores + lax.axis_index("c")
        iota = lax.iota(jnp.int32, 16)
        iota_c = jnp.minimum(iota, d - 1)
        n_uniform = -(-n_blocks // nw)
        n_pairs = -(-n_uniform // 2)

        @pl.when(wid == 0)
        def _():
            pltpu.sync_copy(tail_hbm, tail_v)
            pltpu.sync_copy(tail_v, out_hbm.at[pl.ds(vmain * DP // 128,
                                                     tail * DP // 128), :])

        def body(gg, carry):
            for b in range(2):
                g = gg * 2 + b
                blk = jnp.minimum(wid + g * nw, n_blocks - 1)

                @pl.when(gg > 0)
                def _():
                    pltpu.make_async_copy(
                        out_v.at[b], out_hbm.at[pl.ds(0, rows_blk), :],
                        osems[b]).wait()

                pltpu.async_copy(tt_hbm.at[:, pl.ds(blk * VBLK, VBLK)],
                                 in_v.at[b], isems[b]).wait()

                @plsc.parallel_loop(0, VBLK, 1, unroll=8)
                def vbody(vi, b=b):
                    col = iota * 0 + vi
                    val = plsc.load_gather(in_v.at[b], [iota_c, col])
                    out_v[b, vi // 8, pl.ds((vi % 8) * 16, 16)] = val

                pltpu.async_copy(out_v.at[b],
                                 out_hbm.at[pl.ds(blk * rows_blk, rows_blk),
                                            :],
                                 osems[b])
            return carry

        lax.fori_loop(0, n_pairs, body, 0)
        for b in range(2):
            pltpu.make_async_copy(
                out_v.at[b], out_hbm.at[pl.ds(0, rows_blk), :], osems[b]
            ).wait()

    return detile_kernel


@functools.lru_cache(maxsize=None)
def _make_gather(b_sz: int, l_sz: int, vocab: int, d: int):
    n = b_sz * l_sz
    info = plsc.get_sparse_core_info()
    nw = info.num_cores * info.num_subcores
    assert n % (nw * CHUNK * K) == 0 and b_sz % CHUNK == 0
    assert b_sz % 128 == 0 and l_sz % 8 == 0
    per_w = n // nw
    n_groups = per_w // (CHUNK * K)
    nvec = CHUNK // 16
    bt_per_chunk = CHUNK // 128

    mesh = plsc.VectorSubcoreMesh(core_axis_name="c", subcore_axis_name="s")

    @functools.partial(
        pl.kernel,
        mesh=mesh,
        compiler_params=pltpu.CompilerParams(use_tc_tiling_on_sc=False,
                                             needs_layout_passes=False),
        out_type=jax.ShapeDtypeStruct(
            (d, l_sz // 8, b_sz // 128, 8, 128), jnp.float32),
        scratch_types=[
            pltpu.VMEM((K, CHUNK), jnp.int32),
            pltpu.VMEM((K, CHUNK, DP), jnp.float32),
            pltpu.VMEM((K, d, bt_per_chunk, 1, 128), jnp.float32),
            [pltpu.SemaphoreType.DMA] * K,
            [pltpu.SemaphoreType.DMA] * K,
        ],
    )
    def gather_kernel(idx_hbm, table_hbm, out_hbm, idx_v, rows_v, planes_v,
                      gsems, osems):
        wid = lax.axis_index("s") * info.num_cores + lax.axis_index("c")
        base = wid * per_w
        iota = lax.iota(jnp.int32, 16)
        cols = [jnp.full((16,), c, jnp.int32) for c in range(d)]

        def drain_outs(b):
            for c in range(d):
                pltpu.make_async_copy(
                    planes_v.at[b, c],
                    out_hbm.at[c, 0, pl.ds(0, bt_per_chunk), pl.ds(0, 1)],
                    osems[b],
                ).wait()

        def group(g, carry):
            goff = base + g * (CHUNK * K)
            gathers = []
            for b in range(K):
                off = goff + b * CHUNK

                @pl.when(g > 0)
                def _():
                    drain_outs(b)

                pltpu.sync_copy(idx_hbm.at[pl.ds(off, CHUNK)], idx_v.at[b])
                gathers.append(
                    pltpu.async_copy(table_hbm.at[idx_v.at[b]], rows_v.at[b],
                                     gsems[b]))
            for b in range(K):
                off = goff + b * CHUNK
                gathers[b].wait()
                li = off // b_sz
                lt = li // 8
                ls = li % 8
                bt0 = (off % b_sz) // 128

                @plsc.parallel_loop(0, nvec, 1, unroll=8)
                def jbody(j, b=b):
                    row_idx = j * 16 + iota
                    btj = j // 8
                    lane0 = (j % 8) * 16
                    for c in range(d):
                        val = plsc.load_gather(rows_v.at[b],
                                               [row_idx, cols[c]])
                        planes_v[b, c, btj, 0, pl.ds(lane0, 16)] = val

                for c in range(d):
                    pltpu.async_copy(
                        planes_v.at[b, c],
                        out_hbm.at[c, lt, pl.ds(bt0, bt_per_chunk),
                                   pl.ds(ls, 1)],
                        osems[b],
                    )
            return carry

        lax.fori_loop(0, n_groups, group, 0)
        for b in range(K):
            drain_outs(b)

    return gather_kernel


def kernel(batch, table):
    b_sz, l_sz = batch.shape
    vocab, d = table.shape
    idx_t = batch.T.reshape(-1).astype(jnp.int32)
    vmain = (vocab // VBLK) * VBLK
    tail = jnp.pad(table[vmain:], ((0, 0), (0, DP - d))).reshape(-1, 128)
    t128 = _make_detile(vocab, d)(table.T, tail)
    table_p = t128.reshape(vocab, DP)
    out5 = _make_gather(b_sz, l_sz, vocab, d)(idx_t, table_p)
    return out5.transpose(2, 4, 1, 3, 0).reshape(b_sz, l_sz, d)


# speedup vs baseline: 1.1209x; 1.1209x over previous
import functools

import jax
import jax.numpy as jnp
from jax import lax
from jax.experimental import pallas as pl
from jax.experimental.pallas import tpu as pltpu
from jax.experimental.pallas import tpu_sc as plsc

DP = 16
K = 4
CHUNK = 1024
VBLK = 1024


@functools.lru_cache(maxsize=None)
def _make_detile(vocab: int, d: int):
    info = plsc.get_sparse_core_info()
    nw = info.num_cores * info.num_subcores
    vmain = (vocab // VBLK) * VBLK
    tail = vocab - vmain
    n_blocks = vmain // VBLK
    rows_out = vocab * DP // 128
    rows_blk = VBLK * DP // 128

    mesh = plsc.VectorSubcoreMesh(core_axis_name="c", subcore_axis_name="s")

    @functools.partial(
        pl.kernel,
        mesh=mesh,
        compiler_params=pltpu.CompilerParams(use_tc_tiling_on_sc=True,
                                             needs_layout_passes=False),
        out_type=jax.ShapeDtypeStruct((rows_out, 128), jnp.float32),
        scratch_types=[
            pltpu.VMEM((2, d, VBLK), jnp.float32),
            pltpu.VMEM((2, rows_blk, 128), jnp.float32),
            pltpu.VMEM((tail * DP // 128, 128), jnp.float32),
            [pltpu.SemaphoreType.DMA] * 2,
            [pltpu.SemaphoreType.DMA] * 2,
        ],
    )
    def detile_kernel(tt_hbm, tail_hbm, out_hbm, in_v, out_v, tail_v,
                      isems, osems):
        wid = lax.axis_index("s") * info.num_cores + lax.axis_index("c")
        iota = lax.iota(jnp.int32, 16)
        iota_c = jnp.minimum(iota, d - 1)
        n_uniform = -(-n_blocks // nw)
        n_pairs = -(-n_uniform // 2)

        @pl.when(wid == 0)
        def _():
            pltpu.sync_copy(tail_hbm, tail_v)
            pltpu.sync_copy(tail_v, out_hbm.at[pl.ds(vmain * DP // 128,
                                                     tail * DP // 128), :])

        def body(gg, carry):
            for b in range(2):
                g = gg * 2 + b
                blk = jnp.minimum(wid + g * nw, n_blocks - 1)

                @pl.when(gg > 0)
                def _():
                    pltpu.make_async_copy(
                        out_v.at[b], out_hbm.at[pl.ds(0, rows_blk), :],
                        osems[b]).wait()

                pltpu.async_copy(tt_hbm.at[:, pl.ds(blk * VBLK, VBLK)],
                                 in_v.at[b], isems[b]).wait()

                @plsc.parallel_loop(0, VBLK, 1, unroll=4)
                def vbody(vi, b=b):
                    col = iota * 0 + vi
                    val = plsc.load_gather(in_v.at[b], [iota_c, col])
                    out_v[b, vi // 8, pl.ds((vi % 8) * 16, 16)] = val

                pltpu.async_copy(out_v.at[b],
                                 out_hbm.at[pl.ds(blk * rows_blk, rows_blk),
                                            :],
                                 osems[b])
            return carry

        lax.fori_loop(0, n_pairs, body, 0)
        for b in range(2):
            pltpu.make_async_copy(
                out_v.at[b], out_hbm.at[pl.ds(0, rows_blk), :], osems[b]
            ).wait()

    return detile_kernel


@functools.lru_cache(maxsize=None)
def _make_gather(b_sz: int, l_sz: int, vocab: int, d: int):
    n = b_sz * l_sz
    info = plsc.get_sparse_core_info()
    nw = info.num_cores * info.num_subcores
    assert n % (nw * CHUNK * K) == 0 and b_sz % CHUNK == 0
    assert b_sz % 128 == 0 and l_sz % 8 == 0
    per_w = n // nw
    n_groups = per_w // (CHUNK * K)
    nvec = CHUNK // 16
    bt_per_chunk = CHUNK // 128

    mesh = plsc.VectorSubcoreMesh(core_axis_name="c", subcore_axis_name="s")

    @functools.partial(
        pl.kernel,
        mesh=mesh,
        compiler_params=pltpu.CompilerParams(use_tc_tiling_on_sc=False,
                                             needs_layout_passes=False),
        out_type=jax.ShapeDtypeStruct(
            (d, l_sz // 8, b_sz // 128, 8, 128), jnp.float32),
        scratch_types=[
            pltpu.VMEM((K, CHUNK), jnp.int32),
            pltpu.VMEM((K, CHUNK, DP), jnp.float32),
            pltpu.VMEM((K, d, bt_per_chunk, 1, 128), jnp.float32),
            [pltpu.SemaphoreType.DMA] * K,
            [pltpu.SemaphoreType.DMA] * K,
        ],
    )
    def gather_kernel(idx_hbm, table_hbm, out_hbm, idx_v, rows_v, planes_v,
                      gsems, osems):
        wid = lax.axis_index("s") * info.num_cores + lax.axis_index("c")
        base = wid * per_w
        iota = lax.iota(jnp.int32, 16)
        cols = [jnp.full((16,), c, jnp.int32) for c in range(d)]

        def drain_outs(b):
            for c in range(d):
                pltpu.make_async_copy(
                    planes_v.at[b, c],
                    out_hbm.at[c, 0, pl.ds(0, bt_per_chunk), pl.ds(0, 1)],
                    osems[b],
                ).wait()

        def group(g, carry):
            goff = base + g * (CHUNK * K)
            gathers = []
            for b in range(K):
                off = goff + b * CHUNK

                @pl.when(g > 0)
                def _():
                    drain_outs(b)

                pltpu.sync_copy(idx_hbm.at[pl.ds(off, CHUNK)], idx_v.at[b])
                gathers.append(
                    pltpu.async_copy(table_hbm.at[idx_v.at[b]], rows_v.at[b],
                                     gsems[b]))
            for b in range(K):
                off = goff + b * CHUNK
                gathers[b].wait()
                li = off // b_sz
                lt = li // 8
                ls = li % 8
                bt0 = (off % b_sz) // 128

                @plsc.parallel_loop(0, nvec, 1, unroll=4)
                def jbody(j, b=b):
                    row_idx = j * 16 + iota
                    btj = j // 8
                    lane0 = (j % 8) * 16
                    for c in range(d):
                        val = plsc.load_gather(rows_v.at[b],
                                               [row_idx, cols[c]])
                        planes_v[b, c, btj, 0, pl.ds(lane0, 16)] = val

                for c in range(d):
                    pltpu.async_copy(
                        planes_v.at[b, c],
                        out_hbm.at[c, lt, pl.ds(bt0, bt_per_chunk),
                                   pl.ds(ls, 1)],
                        osems[b],
                    )
            return carry

        lax.fori_loop(0, n_groups, group, 0)
        for b in range(K):
            drain_outs(b)

    return gather_kernel


def kernel(batch, table):
    b_sz, l_sz = batch.shape
    vocab, d = table.shape
    idx_t = batch.T.reshape(-1).astype(jnp.int32)
    vmain = (vocab // VBLK) * VBLK
    tail = jnp.pad(table[vmain:], ((0, 0), (0, DP - d))).reshape(-1, 128)
    t128 = _make_detile(vocab, d)(table.T, tail)
    table_p = t128.reshape(vocab, DP)
    out5 = _make_gather(b_sz, l_sz, vocab, d)(idx_t, table_p)
    return out5.transpose(2, 4, 1, 3, 0).reshape(b_sz, l_sz, d)
